# baseline (speedup 1.0000x reference)
import functools

import jax
import jax.numpy as jnp
from jax import lax
from jax.experimental import pallas as pl
from jax.experimental.pallas import tpu as pltpu
from jax.experimental.pallas import tpu_sc as plsc

NN = 2048
NE = 16384
P = 2 * NE
D = 64
B = 4

NC, NS, L = 2, 16, 16
CB = 128
PPS = P // NS
NCH = PPS // CB
BPC = B // NC
RPS = NN // NS
CPB = P // CB
OB = P + NN

_SC_PARAMS = pltpu.CompilerParams(use_tc_tiling_on_sc=False,
                                  needs_layout_passes=False)
_MESH = plsc.VectorSubcoreMesh(core_axis_name="c", subcore_axis_name="s",
                               num_cores=NC, num_subcores=NS)


def _dense_body(x_ref, we_ref, wn_ref, bn_ref, wc_ref, bc_ref,
                s2_ref, d2_ref, p2_ref,
                h_ref, z1_ref, ze_ref, z3_ref, idx_ref):
    xb = x_ref[0]
    edges = xb[:NE]
    nodes = xb[NE:]
    W1 = wc_ref[0:D]
    W2 = wc_ref[D:2 * D]
    W3 = wc_ref[2 * D:3 * D]
    Wn = wn_ref[...]
    We = we_ref[...]
    bn = bn_ref[...][None, :]
    bc = bc_ref[...][None, :]
    dot = functools.partial(jnp.dot, preferred_element_type=jnp.float32)
    G1 = dot(Wn, W1)
    G3 = dot(Wn, W3)
    Ge = dot(We, W2)
    bias_f = dot(bn, W1 + W3) + bc
    h_ref[...] = dot(nodes, Wn) + bn
    z1_ref[...] = dot(nodes, G1) + bias_f
    z3_ref[...] = dot(nodes, G3)
    ze_ref[...] = dot(edges, Ge)
    b = pl.program_id(0)
    idx_ref[...] = jnp.concatenate(
        (s2_ref[...] + b * NN, p2_ref[...] + b * NE, d2_ref[...] + b * NN),
        axis=1)


def _dense_stage(x, W_edge, W_node, b_node, W_comb, b_comb, s2, d2, p2):
    return pl.pallas_call(
        _dense_body,
        grid=(B,),
        in_specs=[
            pl.BlockSpec((1, NE + NN, D), lambda b: (b, 0, 0)),
            pl.BlockSpec((D, D), lambda b: (0, 0)),
            pl.BlockSpec((D, D), lambda b: (0, 0)),
            pl.BlockSpec((D,), lambda b: (0,)),
            pl.BlockSpec((3 * D, D), lambda b: (0, 0)),
            pl.BlockSpec((D,), lambda b: (0,)),
            pl.BlockSpec((CPB, CB), lambda b: (0, 0)),
            pl.BlockSpec((CPB, CB), lambda b: (0, 0)),
            pl.BlockSpec((CPB, CB), lambda b: (0, 0)),
        ],
        out_specs=[
            pl.BlockSpec((NN, D), lambda b: (b, 0)),
            pl.BlockSpec((NN, D), lambda b: (b, 0)),
            pl.BlockSpec((NE, D), lambda b: (b, 0)),
            pl.BlockSpec((NN, D), lambda b: (b, 0)),
            pl.BlockSpec((CPB, 3 * CB), lambda b: (b, 0)),
        ],
        out_shape=[
            jax.ShapeDtypeStruct((B * NN, D), jnp.float32),
            jax.ShapeDtypeStruct((B * NN, D), jnp.float32),
            jax.ShapeDtypeStruct((B * NE, D), jnp.float32),
            jax.ShapeDtypeStruct((B * NN, D), jnp.float32),
            jax.ShapeDtypeStruct((B * CPB, 3 * CB), jnp.int32),
        ],
    )(x, W_edge, W_node, b_node, W_comb, b_comb, s2, d2, p2)


def _fused_body(z1_hbm, ze_hbm, z3_hbm, h_hbm, idx_hbm, src2_hbm, w_hbm,
                out_hbm, idxa_v, srca_v, w_v, accd_v, t_v, stat_v, red_v,
                bufs, acc_sh, mred_sh, sred_sh, sems):
    cid = lax.axis_index("c")
    sid = lax.axis_index("s")
    (r10, r20, r30, o0, r11, r21, r31, o1) = bufs
    (semg0, semg1, semst0, semst1, semsc0, semsc1) = sems
    gsets = ((r10, r20, r30, o0, semg0, semst0, semsc0),
             (r11, r21, r31, o1, semg1, semst1, semsc1))
    iota = lax.iota(jnp.int32, L)

    pltpu.sync_copy(src2_hbm.at[pl.ds(sid * NCH, NCH), :], srca_v)
    pltpu.sync_copy(w_hbm, w_v)

    def issue1(setid, k):
        r1, r2, r3, _o, semg, _s, _sc = gsets[setid]
        pltpu.async_copy(z1_hbm.at[idxa_v.at[k, pl.ds(0, CB)]], r1, semg)
        pltpu.async_copy(ze_hbm.at[idxa_v.at[k, pl.ds(CB, CB)]], r2, semg)
        pltpu.async_copy(z3_hbm.at[idxa_v.at[k, pl.ds(2 * CB, CB)]], r3, semg)

    def process1(setid, k, obase, m16):
        r1, r2, r3, o, semg, semst, _sc = gsets[setid]
        pltpu.make_async_copy(z1_hbm.at[pl.ds(0, CB)], r1, semg).wait()
        pltpu.make_async_copy(z1_hbm.at[pl.ds(0, CB)], r2, semg).wait()
        pltpu.make_async_copy(z1_hbm.at[pl.ds(0, CB)], r3, semg).wait()

        @pl.when(k >= 2)
        def _():
            pltpu.make_async_copy(o, out_hbm.at[pl.ds(0, CB)], semst).wait()

        def row(i, c):
            acc = jnp.zeros((L,), jnp.float32)
            for dch in range(D // L):
                sl = pl.ds(dch * L, L)
                v = r1[i, sl] + r2[i, sl] + r3[i, sl]
                v = jnp.where(v >= 0.0, v, 0.01 * v)
                o[i, sl] = v
                acc = acc + v * w_v[sl]
            accd_v[i, :] = acc
            return c

        lax.fori_loop(0, CB, row, 0, unroll=4)
        pltpu.async_copy(o, out_hbm.at[pl.ds(obase + k * CB, CB)], semst)

        def grp(g, m):
            ridx = g * L + iota
            t16 = plsc.load_gather(
                accd_v, [ridx, jnp.zeros((L,), jnp.int32)])
            for j in range(1, L):
                t16 = t16 + plsc.load_gather(
                    accd_v, [ridx, jnp.full((L,), j, jnp.int32)])
            t_v[pl.ds(k * CB + g * L, L)] = t16
            return jnp.maximum(m, t16)

        return lax.fori_loop(0, CB // L, grp, m16, unroll=2)

    def issue2(setid, k):
        rw, _r2, _r3, _o, semg, _s, _sc = gsets[setid]
        pltpu.async_copy(h_hbm.at[idxa_v.at[k, pl.ds(2 * CB, CB)]], rw, semg)

    def process2(setid, k, rcp16):
        rw, _r2, _r3, _o, semg, _s, semsc = gsets[setid]
        pltpu.make_async_copy(h_hbm.at[pl.ds(0, CB)], rw, semg).wait()

        def row(i, c):
            g = plsc.load_gather(t_v, [jnp.broadcast_to(k * CB + i, (L,))])
            g = g * rcp16
            for dch in range(D // L):
                sl = pl.ds(dch * L, L)
                rw[i, sl] = rw[i, sl] * g
            return c

        lax.fori_loop(0, CB, row, 0, unroll=4)
        pltpu.async_copy(rw, acc_sh.at[srca_v.at[k]], semsc, add=True)

    def drain_scatter(setid):
        rw, _r2, _r3, _o, _g, _s, semsc = gsets[setid]
        pltpu.make_async_copy(rw, acc_sh.at[pl.ds(0, CB)], semsc).wait()

    def zero_row(i, c):
        for dch in range(D // L):
            o0[i, pl.ds(dch * L, L)] = jnp.zeros((L,), jnp.float32)
        return c

    for b_loc in range(BPC):
        bg = cid * BPC + b_loc
        pltpu.sync_copy(
            idx_hbm.at[pl.ds(bg * CPB + sid * NCH, NCH), :], idxa_v)
        obase = bg * OB + sid * PPS

        issue1(0, 0)
        issue1(1, 1)

        def step1(k, m16):
            m16 = process1(0, 2 * k, obase, m16)

            @pl.when(2 * k + 2 < NCH)
            def _():
                issue1(0, 2 * k + 2)

            m16 = process1(1, 2 * k + 1, obase, m16)

            @pl.when(2 * k + 3 < NCH)
            def _():
                issue1(1, 2 * k + 3)

            return m16

        m16 = lax.fori_loop(0, NCH // 2, step1,
                            jnp.full((L,), -jnp.inf, jnp.float32))
        pltpu.make_async_copy(o0, out_hbm.at[pl.ds(0, CB)], semst0).wait()
        pltpu.make_async_copy(o1, out_hbm.at[pl.ds(0, CB)], semst1).wait()

        stat_v[...] = m16
        pltpu.sync_copy(stat_v, mred_sh.at[sid])
        lax.fori_loop(0, RPS, zero_row, 0)
        pltpu.sync_copy(o0.at[pl.ds(0, RPS)], acc_sh.at[pl.ds(sid * RPS, RPS)])
        plsc.subcore_barrier()
        pltpu.sync_copy(mred_sh, red_v)

        def redmax(i, m):
            return jnp.maximum(m, red_v[i, :])

        gmax = lax.fori_loop(0, NS, redmax,
                             jnp.full((L,), -jnp.inf, jnp.float32))
        M16 = jnp.broadcast_to(jnp.max(gmax), (L,))

        def expgrp(j, s):
            sl = pl.ds(j * L, L)
            e = jnp.exp(t_v[sl] - M16)
            t_v[sl] = e
            return s + e

        s16 = lax.fori_loop(0, PPS // L, expgrp,
                            jnp.zeros((L,), jnp.float32), unroll=4)
        stat_v[...] = s16
        pltpu.sync_copy(stat_v, sred_sh.at[sid])
        plsc.subcore_barrier()
        pltpu.sync_copy(sred_sh, red_v)

        def redsum(i, s):
            return s + red_v[i, :]

        gsum = lax.fori_loop(0, NS, redsum, jnp.zeros((L,), jnp.float32))
        rcp16 = 1.0 / jnp.broadcast_to(jnp.sum(gsum), (L,))

        issue2(0, 0)
        issue2(1, 1)

        def step2(k, c):
            process2(0, 2 * k, rcp16)
            process2(1, 2 * k + 1, rcp16)

            @pl.when(2 * k + 2 < NCH)
            def _():
                drain_scatter(0)
                issue2(0, 2 * k + 2)

            @pl.when(2 * k + 3 < NCH)
            def _():
                drain_scatter(1)
                issue2(1, 2 * k + 3)

            return c

        lax.fori_loop(0, NCH // 2, step2, jnp.int32(0))
        drain_scatter(0)
        drain_scatter(1)
        plsc.subcore_barrier()
        pltpu.sync_copy(acc_sh.at[pl.ds(sid * RPS, RPS)],
                        out_hbm.at[pl.ds(bg * OB + P + sid * RPS, RPS)])
        plsc.subcore_barrier()


_fused_call = pl.kernel(
    _fused_body,
    out_type=jax.ShapeDtypeStruct((B * OB, D), jnp.float32),
    mesh=_MESH,
    scratch_types=[
        pltpu.VMEM((NCH, 3 * CB), jnp.int32),
        pltpu.VMEM((NCH, CB), jnp.int32),
        pltpu.VMEM((D,), jnp.float32),
        pltpu.VMEM((CB, L), jnp.float32),
        pltpu.VMEM((PPS,), jnp.float32),
        pltpu.VMEM((L,), jnp.float32),
        pltpu.VMEM((NS, L), jnp.float32),
        tuple([pltpu.VMEM((CB, D), jnp.float32)] * 8),
        pltpu.VMEM_SHARED((NN, D), jnp.float32),
        pltpu.VMEM_SHARED((NS, L), jnp.float32),
        pltpu.VMEM_SHARED((NS, L), jnp.float32),
        tuple([pltpu.SemaphoreType.DMA] * 6),
    ],
    compiler_params=_SC_PARAMS,
)


def kernel(x, W_edge, W_node, b_node, W_comb, b_comb, W_attn,
           src, dst, pair_edge):
    s2 = src.reshape(CPB, CB)
    d2 = dst.reshape(CPB, CB)
    p2 = pair_edge.reshape(CPB, CB)
    h, z1, ze, z3, idx = _dense_stage(
        x, W_edge, W_node, b_node, W_comb, b_comb, s2, d2, p2)
    out = _fused_call(z1, ze, z3, h, idx, s2, W_attn.reshape(D))
    return out.reshape(B, OB, D)


# speedup vs baseline: 1.1635x; 1.1635x over previous
import functools

import jax
import jax.numpy as jnp
from jax import lax
from jax.experimental import pallas as pl
from jax.experimental.pallas import tpu as pltpu
from jax.experimental.pallas import tpu_sc as plsc

NN = 2048
NE = 16384
P = 2 * NE
D = 64
B = 4

NC, NS, L = 2, 16, 16
CB = 128
PPS = P // NS
NCH = PPS // CB
BPC = B // NC
RPS = NN // NS
CPB = P // CB
OB = P + NN

_SC_PARAMS = pltpu.CompilerParams(use_tc_tiling_on_sc=False,
                                  needs_layout_passes=False)
_MESH = plsc.VectorSubcoreMesh(core_axis_name="c", subcore_axis_name="s",
                               num_cores=NC, num_subcores=NS)


def _dense_body(x_ref, we_ref, wn_ref, bn_ref, wc_ref, bc_ref,
                s2_ref, d2_ref, p2_ref,
                h_ref, z1_ref, ze_ref, z3_ref, idx_ref):
    xb = x_ref[0]
    edges = xb[:NE]
    nodes = xb[NE:]
    W1 = wc_ref[0:D]
    W2 = wc_ref[D:2 * D]
    W3 = wc_ref[2 * D:3 * D]
    Wn = wn_ref[...]
    We = we_ref[...]
    bn = bn_ref[...][None, :]
    bc = bc_ref[...][None, :]
    dot = functools.partial(jnp.dot, preferred_element_type=jnp.float32)
    G1 = dot(Wn, W1)
    G3 = dot(Wn, W3)
    Ge = dot(We, W2)
    bias_f = dot(bn, W1 + W3) + bc
    h_ref[...] = dot(nodes, Wn) + bn
    z1_ref[...] = dot(nodes, G1) + bias_f
    z3_ref[...] = dot(nodes, G3)
    ze_ref[...] = dot(edges, Ge)
    b = pl.program_id(0)
    idx_ref[...] = jnp.concatenate(
        (s2_ref[...] + b * NN, p2_ref[...] + b * NE, d2_ref[...] + b * NN),
        axis=1)


def _dense_stage(x, W_edge, W_node, b_node, W_comb, b_comb, s2, d2, p2):
    return pl.pallas_call(
        _dense_body,
        grid=(B,),
        in_specs=[
            pl.BlockSpec((1, NE + NN, D), lambda b: (b, 0, 0)),
            pl.BlockSpec((D, D), lambda b: (0, 0)),
            pl.BlockSpec((D, D), lambda b: (0, 0)),
            pl.BlockSpec((D,), lambda b: (0,)),
            pl.BlockSpec((3 * D, D), lambda b: (0, 0)),
            pl.BlockSpec((D,), lambda b: (0,)),
            pl.BlockSpec((CPB, CB), lambda b: (0, 0)),
            pl.BlockSpec((CPB, CB), lambda b: (0, 0)),
            pl.BlockSpec((CPB, CB), lambda b: (0, 0)),
        ],
        out_specs=[
            pl.BlockSpec((NN, D), lambda b: (b, 0)),
            pl.BlockSpec((NN, D), lambda b: (b, 0)),
            pl.BlockSpec((NE, D), lambda b: (b, 0)),
            pl.BlockSpec((NN, D), lambda b: (b, 0)),
            pl.BlockSpec((CPB, 3 * CB), lambda b: (b, 0)),
        ],
        out_shape=[
            jax.ShapeDtypeStruct((B * NN, D), jnp.float32),
            jax.ShapeDtypeStruct((B * NN, D), jnp.float32),
            jax.ShapeDtypeStruct((B * NE, D), jnp.float32),
            jax.ShapeDtypeStruct((B * NN, D), jnp.float32),
            jax.ShapeDtypeStruct((B * CPB, 3 * CB), jnp.int32),
        ],
    )(x, W_edge, W_node, b_node, W_comb, b_comb, s2, d2, p2)


def _fused_body(z1_hbm, ze_hbm, z3_hbm, h_hbm, idx_hbm, src2_hbm, w_hbm,
                out_hbm, idxa_v, srca_v, w_v, accd_v, t_v, stat_v, red_v,
                bufs, acc_sh, mred_sh, sred_sh, sems):
    cid = lax.axis_index("c")
    sid = lax.axis_index("s")
    (r10, r20, r30, o0, r11, r21, r31, o1) = bufs
    (semg0, semg1, semst0, semst1, semsc0, semsc1) = sems
    gsets = ((r10, r20, r30, o0, semg0, semst0, semsc0),
             (r11, r21, r31, o1, semg1, semst1, semsc1))
    iota = lax.iota(jnp.int32, L)

    pltpu.sync_copy(src2_hbm.at[pl.ds(sid * NCH, NCH), :], srca_v)
    pltpu.sync_copy(w_hbm, w_v)

    def issue1(setid, k):
        r1, r2, r3, _o, semg, _s, _sc = gsets[setid]
        pltpu.async_copy(z1_hbm.at[idxa_v.at[k, pl.ds(0, CB)]], r1, semg)
        pltpu.async_copy(ze_hbm.at[idxa_v.at[k, pl.ds(CB, CB)]], r2, semg)
        pltpu.async_copy(z3_hbm.at[idxa_v.at[k, pl.ds(2 * CB, CB)]], r3, semg)

    def process1(setid, k, obase, m16):
        r1, r2, r3, o, semg, semst, _sc = gsets[setid]
        pltpu.make_async_copy(z1_hbm.at[pl.ds(0, CB)], r1, semg).wait()
        pltpu.make_async_copy(z1_hbm.at[pl.ds(0, CB)], r2, semg).wait()
        pltpu.make_async_copy(z1_hbm.at[pl.ds(0, CB)], r3, semg).wait()

        @pl.when(k >= 2)
        def _():
            pltpu.make_async_copy(o, out_hbm.at[pl.ds(0, CB)], semst).wait()

        def row(i, c):
            acc = jnp.zeros((L,), jnp.float32)
            for dch in range(D // L):
                sl = pl.ds(dch * L, L)
                v = r1[i, sl] + r2[i, sl] + r3[i, sl]
                v = jnp.where(v >= 0.0, v, 0.01 * v)
                o[i, sl] = v
                acc = acc + v * w_v[sl]
            accd_v[i, :] = acc
            return c

        lax.fori_loop(0, CB, row, 0)
        pltpu.async_copy(o, out_hbm.at[pl.ds(obase + k * CB, CB)], semst)

        def grp(g, m):
            ridx = g * L + iota
            t16 = plsc.load_gather(
                accd_v, [ridx, jnp.zeros((L,), jnp.int32)])
            for j in range(1, L):
                t16 = t16 + plsc.load_gather(
                    accd_v, [ridx, jnp.full((L,), j, jnp.int32)])
            t_v[pl.ds(k * CB + g * L, L)] = t16
            return jnp.maximum(m, t16)

        return lax.fori_loop(0, CB // L, grp, m16)

    def issue2(setid, k):
        rw, _r2, _r3, _o, semg, _s, _sc = gsets[setid]
        pltpu.async_copy(h_hbm.at[idxa_v.at[k, pl.ds(2 * CB, CB)]], rw, semg)

    def process2(setid, k, rcp16):
        rw, _r2, _r3, _o, semg, _s, semsc = gsets[setid]
        pltpu.make_async_copy(h_hbm.at[pl.ds(0, CB)], rw, semg).wait()

        def row(i, c):
            g = plsc.load_gather(t_v, [jnp.broadcast_to(k * CB + i, (L,))])
            g = g * rcp16
            for dch in range(D // L):
                sl = pl.ds(dch * L, L)
                rw[i, sl] = rw[i, sl] * g
            return c

        lax.fori_loop(0, CB, row, 0)
        pltpu.async_copy(rw, acc_sh.at[srca_v.at[k]], semsc, add=True)

    def drain_scatter(setid):
        rw, _r2, _r3, _o, _g, _s, semsc = gsets[setid]
        pltpu.make_async_copy(rw, acc_sh.at[pl.ds(0, CB)], semsc).wait()

    def zero_row(i, c):
        for dch in range(D // L):
            o0[i, pl.ds(dch * L, L)] = jnp.zeros((L,), jnp.float32)
        return c

    for b_loc in range(BPC):
        bg = cid * BPC + b_loc
        pltpu.sync_copy(
            idx_hbm.at[pl.ds(bg * CPB + sid * NCH, NCH), :], idxa_v)
        obase = bg * OB + sid * PPS

        issue1(0, 0)
        issue1(1, 1)

        def step1(k, m16):
            m16 = process1(0, 2 * k, obase, m16)

            @pl.when(2 * k + 2 < NCH)
            def _():
                issue1(0, 2 * k + 2)

            m16 = process1(1, 2 * k + 1, obase, m16)

            @pl.when(2 * k + 3 < NCH)
            def _():
                issue1(1, 2 * k + 3)

            return m16

        m16 = lax.fori_loop(0, NCH // 2, step1,
                            jnp.full((L,), -jnp.inf, jnp.float32))
        pltpu.make_async_copy(o0, out_hbm.at[pl.ds(0, CB)], semst0).wait()
        pltpu.make_async_copy(o1, out_hbm.at[pl.ds(0, CB)], semst1).wait()

        stat_v[...] = m16
        pltpu.sync_copy(stat_v, mred_sh.at[sid])
        lax.fori_loop(0, RPS, zero_row, 0)
        pltpu.sync_copy(o0.at[pl.ds(0, RPS)], acc_sh.at[pl.ds(sid * RPS, RPS)])
        plsc.subcore_barrier()
        pltpu.sync_copy(mred_sh, red_v)

        def redmax(i, m):
            return jnp.maximum(m, red_v[i, :])

        gmax = lax.fori_loop(0, NS, redmax,
                             jnp.full((L,), -jnp.inf, jnp.float32))
        M16 = jnp.broadcast_to(jnp.max(gmax), (L,))

        def expgrp(j, s):
            sl = pl.ds(j * L, L)
            e = jnp.exp(t_v[sl] - M16)
            t_v[sl] = e
            return s + e

        s16 = lax.fori_loop(0, PPS // L, expgrp, jnp.zeros((L,), jnp.float32))
        stat_v[...] = s16
        pltpu.sync_copy(stat_v, sred_sh.at[sid])
        plsc.subcore_barrier()
        pltpu.sync_copy(sred_sh, red_v)

        def redsum(i, s):
            return s + red_v[i, :]

        gsum = lax.fori_loop(0, NS, redsum, jnp.zeros((L,), jnp.float32))
        rcp16 = 1.0 / jnp.broadcast_to(jnp.sum(gsum), (L,))

        issue2(0, 0)
        issue2(1, 1)

        def step2(k, c):
            process2(0, 2 * k, rcp16)
            process2(1, 2 * k + 1, rcp16)

            @pl.when(2 * k + 2 < NCH)
            def _():
                drain_scatter(0)
                issue2(0, 2 * k + 2)

            @pl.when(2 * k + 3 < NCH)
            def _():
                drain_scatter(1)
                issue2(1, 2 * k + 3)

            return c

        lax.fori_loop(0, NCH // 2, step2, jnp.int32(0))
        drain_scatter(0)
        drain_scatter(1)
        plsc.subcore_barrier()
        pltpu.sync_copy(acc_sh.at[pl.ds(sid * RPS, RPS)],
                        out_hbm.at[pl.ds(bg * OB + P + sid * RPS, RPS)])
        plsc.subcore_barrier()


_fused_call = pl.kernel(
    _fused_body,
    out_type=jax.ShapeDtypeStruct((B * OB, D), jnp.float32),
    mesh=_MESH,
    scratch_types=[
        pltpu.VMEM((NCH, 3 * CB), jnp.int32),
        pltpu.VMEM((NCH, CB), jnp.int32),
        pltpu.VMEM((D,), jnp.float32),
        pltpu.VMEM((CB, L), jnp.float32),
        pltpu.VMEM((PPS,), jnp.float32),
        pltpu.VMEM((L,), jnp.float32),
        pltpu.VMEM((NS, L), jnp.float32),
        tuple([pltpu.VMEM((CB, D), jnp.float32)] * 8),
        pltpu.VMEM_SHARED((NN, D), jnp.float32),
        pltpu.VMEM_SHARED((NS, L), jnp.float32),
        pltpu.VMEM_SHARED((NS, L), jnp.float32),
        tuple([pltpu.SemaphoreType.DMA] * 6),
    ],
    compiler_params=_SC_PARAMS,
)


def kernel(x, W_edge, W_node, b_node, W_comb, b_comb, W_attn,
           src, dst, pair_edge):
    s2 = src.reshape(CPB, CB)
    d2 = dst.reshape(CPB, CB)
    p2 = pair_edge.reshape(CPB, CB)
    h, z1, ze, z3, idx = _dense_stage(
        x, W_edge, W_node, b_node, W_comb, b_comb, s2, d2, p2)
    out = _fused_call(z1, ze, z3, h, idx, s2, W_attn.reshape(D))
    return out.reshape(B, OB, D)
